# baseline (device time: 26002 ns/iter reference)
import jax
import jax.numpy as jnp
from jax import lax
from jax.experimental import pallas as pl
from jax.experimental.pallas import tpu as pltpu


def kernel(Q, K, V, bt, lens):
    B, QL, H, D = Q.shape
    P_loc, BS, _, _ = K.shape
    NB = bt.shape[1]
    scale = D ** -0.5

    def body(q_ref, k_ref, v_ref, bt_ref, lens_ref, out_ref,
             o_send, o_recv, st_send, st_recv, send_sems, recv_sems):
        my_x = lax.axis_index("x")
        my_y = lax.axis_index("y")
        my_z = lax.axis_index("z")
        partner = (my_x, my_y, 1 - my_z)

        barrier_sem = pltpu.get_barrier_semaphore()
        pl.semaphore_signal(barrier_sem, inc=1, device_id=partner,
                            device_id_type=pl.DeviceIdType.MESH)

        z_off = my_z * P_loc
        btv = bt_ref[:, :]
        lensv = lens_ref[:, :]
        p_iota = lax.broadcasted_iota(jnp.int32, (B, NB, P_loc), 2) + z_off
        j_iota = lax.broadcasted_iota(jnp.int32, (B, NB, P_loc), 1)
        hit = (btv[:, :, None] == p_iota) & (j_iota < lensv[:, :, None])
        w_page = jnp.sum(hit.astype(jnp.float32), axis=1)
        w_pos = w_page > 0
        neg = jnp.float32(-1e30)

        qT = jnp.transpose(q_ref[:, 0, :, :], (1, 0, 2))
        qb = jnp.broadcast_to(qT[None], (BS, H, B, D)).reshape(BS * H, B, D)
        k_all = k_ref[:, :, :, :].reshape(BS * H, D, P_loc)
        v_all = v_ref[:, :, :, :].reshape(BS * H, D, P_loc)
        s3 = lax.dot_general(
            qb, k_all, (((2,), (1,)), ((0,), (0,))),
            preferred_element_type=jnp.float32) * scale
        s4 = jnp.where(w_pos[None, None, :, :],
                       s3.reshape(BS, H, B, P_loc), neg)
        m1 = jnp.max(s4, axis=0)
        m = jnp.max(m1, axis=2)
        p4 = jnp.exp(s4 - m[None, :, :, None]) * w_page[None, None, :, :]
        l = jnp.sum(jnp.sum(p4, axis=0), axis=2)
        o3 = lax.dot_general(
            p4.reshape(BS * H, B, P_loc), v_all, (((2,), (2,)), ((0,), (0,))),
            preferred_element_type=jnp.float32)
        o = jnp.sum(o3.reshape(BS, H, B, D), axis=0)
        o_send[:, :, :] = o
        st_send[:, :, 0:1] = m[:, :, None]
        st_send[:, :, 1:2] = l[:, :, None]

        pl.semaphore_wait(barrier_sem, 1)

        rdma_o = pltpu.make_async_remote_copy(
            src_ref=o_send, dst_ref=o_recv,
            send_sem=send_sems.at[0], recv_sem=recv_sems.at[0],
            device_id=partner, device_id_type=pl.DeviceIdType.MESH)
        rdma_st = pltpu.make_async_remote_copy(
            src_ref=st_send, dst_ref=st_recv,
            send_sem=send_sems.at[1], recv_sem=recv_sems.at[1],
            device_id=partner, device_id_type=pl.DeviceIdType.MESH)
        rdma_o.start()
        rdma_st.start()
        rdma_o.wait()
        rdma_st.wait()

        m_r = st_recv[:, :, 0]
        l_r = st_recv[:, :, 1]
        m_n = jnp.maximum(m, m_r)
        a_l = jnp.exp(m - m_n)
        a_r = jnp.exp(m_r - m_n)
        l_tot = a_l * l + a_r * l_r
        comb = (a_l[:, :, None] * o + a_r[:, :, None] * o_recv[:, :, :]) \
            / l_tot[:, :, None]
        out_ref[:, 0, :, :] = jnp.transpose(comb, (1, 0, 2))

    K2 = jnp.transpose(K, (1, 2, 3, 0))
    V2 = jnp.transpose(V, (1, 2, 3, 0))

    return pl.pallas_call(
        body,
        out_shape=jax.ShapeDtypeStruct((B, QL, H, D), jnp.float32),
        in_specs=[pl.BlockSpec(memory_space=pltpu.VMEM)] * 5,
        out_specs=pl.BlockSpec(memory_space=pltpu.VMEM),
        scratch_shapes=[
            pltpu.VMEM((H, B, D), jnp.float32),
            pltpu.VMEM((H, B, D), jnp.float32),
            pltpu.VMEM((H, B, 2), jnp.float32),
            pltpu.VMEM((H, B, 2), jnp.float32),
            pltpu.SemaphoreType.DMA((2,)),
            pltpu.SemaphoreType.DMA((2,)),
        ],
        compiler_params=pltpu.CompilerParams(
            collective_id=0, vmem_limit_bytes=48 * 1024 * 1024),
    )(Q, K2, V2, bt, lens.reshape(B, 1))


# device time: 19731 ns/iter; 1.3178x vs baseline; 1.3178x over previous
import jax
import jax.numpy as jnp
from jax import lax
from jax.experimental import pallas as pl
from jax.experimental.pallas import tpu as pltpu


def kernel(Q, K, V, bt, lens):
    B, QL, H, D = Q.shape
    P_loc, BS, _, _ = K.shape
    NB = bt.shape[1]
    scale = D ** -0.5

    def body(q_ref, k_ref, v_ref, bt_ref, lens_ref, out_ref,
             o_send, o_recv, st_send, st_recv, send_sems, recv_sems):
        my_x = lax.axis_index("x")
        my_y = lax.axis_index("y")
        my_z = lax.axis_index("z")
        partner = (my_x, my_y, 1 - my_z)

        barrier_sem = pltpu.get_barrier_semaphore()
        pl.semaphore_signal(barrier_sem, inc=1, device_id=partner,
                            device_id_type=pl.DeviceIdType.MESH)

        z_off = my_z * P_loc
        btv = bt_ref[:, :]
        lensv = lens_ref[:, :]
        p_iota = lax.broadcasted_iota(jnp.int32, (B, NB, P_loc), 2) + z_off
        j_iota = lax.broadcasted_iota(jnp.int32, (B, NB, P_loc), 1)
        hit = (btv[:, :, None] == p_iota) & (j_iota < lensv[:, :, None])
        w_page = jnp.sum(hit.astype(jnp.float32), axis=1)
        lnw = jnp.where(w_page > 0, jnp.log(w_page), jnp.float32(-1e30))

        qT = jnp.transpose(q_ref[:, 0, :, :], (1, 0, 2)) * scale
        qb = jnp.broadcast_to(qT[None], (BS, H, B, D)).reshape(BS * H, B, D)
        k_all = k_ref[:, :, :, :].reshape(BS * H, D, P_loc)
        v_all = v_ref[:, :, :, :].reshape(BS * H, D, P_loc)
        s3 = lax.dot_general(
            qb, k_all, (((2,), (1,)), ((0,), (0,))),
            preferred_element_type=jnp.float32)
        s4 = s3.reshape(BS, H, B, P_loc) + lnw[None, None, :, :]
        m1 = jnp.max(s4, axis=0)
        m = jnp.max(m1, axis=2)
        p4 = jnp.exp(s4 - m[None, :, :, None])
        l = jnp.sum(jnp.sum(p4, axis=0), axis=2)
        o3 = lax.dot_general(
            p4.reshape(BS * H, B, P_loc), v_all, (((2,), (2,)), ((0,), (0,))),
            preferred_element_type=jnp.float32)
        o = jnp.sum(o3.reshape(BS, H, B, D), axis=0)
        o_send[:, :, :] = o
        st_send[:, :, 0:1] = m[:, :, None]
        st_send[:, :, 1:2] = l[:, :, None]

        pl.semaphore_wait(barrier_sem, 1)

        rdma_o = pltpu.make_async_remote_copy(
            src_ref=o_send, dst_ref=o_recv,
            send_sem=send_sems.at[0], recv_sem=recv_sems.at[0],
            device_id=partner, device_id_type=pl.DeviceIdType.MESH)
        rdma_st = pltpu.make_async_remote_copy(
            src_ref=st_send, dst_ref=st_recv,
            send_sem=send_sems.at[1], recv_sem=recv_sems.at[1],
            device_id=partner, device_id_type=pl.DeviceIdType.MESH)
        rdma_o.start()
        rdma_st.start()
        rdma_o.wait()
        rdma_st.wait()

        m_r = st_recv[:, :, 0]
        l_r = st_recv[:, :, 1]
        m_n = jnp.maximum(m, m_r)
        a_l = jnp.exp(m - m_n)
        a_r = jnp.exp(m_r - m_n)
        l_tot = a_l * l + a_r * l_r
        comb = (a_l[:, :, None] * o + a_r[:, :, None] * o_recv[:, :, :]) \
            / l_tot[:, :, None]
        out_ref[:, 0, :, :] = jnp.transpose(comb, (1, 0, 2))

    K2 = jnp.transpose(K, (1, 2, 3, 0))
    V2 = jnp.transpose(V, (1, 2, 3, 0))

    return pl.pallas_call(
        body,
        out_shape=jax.ShapeDtypeStruct((B, QL, H, D), jnp.float32),
        in_specs=[pl.BlockSpec(memory_space=pltpu.VMEM)] * 5,
        out_specs=pl.BlockSpec(memory_space=pltpu.VMEM),
        scratch_shapes=[
            pltpu.VMEM((H, B, D), jnp.float32),
            pltpu.VMEM((H, B, D), jnp.float32),
            pltpu.VMEM((H, B, 2), jnp.float32),
            pltpu.VMEM((H, B, 2), jnp.float32),
            pltpu.SemaphoreType.DMA((2,)),
            pltpu.SemaphoreType.DMA((2,)),
        ],
        compiler_params=pltpu.CompilerParams(
            collective_id=0, vmem_limit_bytes=48 * 1024 * 1024),
    )(Q, K2, V2, bt, lens.reshape(B, 1))


# device time: 15518 ns/iter; 1.6756x vs baseline; 1.2715x over previous
import jax
import jax.numpy as jnp
from jax import lax
from jax.experimental import pallas as pl
from jax.experimental.pallas import tpu as pltpu

N_CHUNK = 8


def kernel(Q, K, V, bt, lens):
    B, QL, H, D = Q.shape
    P_loc, BS, _, _ = K.shape
    NB = bt.shape[1]
    HC = H // N_CHUNK
    scale = D ** -0.5

    def body(q_ref, k_ref, v_ref, bt_ref, lens_ref, out_ref,
             kq, vq, o_send, o_recv,
             dma_sems, send_sems, recv_sems):
        my_x = lax.axis_index("x")
        my_y = lax.axis_index("y")
        my_z = lax.axis_index("z")
        partner = (my_x, my_y, 1 - my_z)

        barrier_sem = pltpu.get_barrier_semaphore()
        pl.semaphore_signal(barrier_sem, inc=1, device_id=partner,
                            device_id_type=pl.DeviceIdType.MESH)

        dmas = []
        for c in range(N_CHUNK):
            h0 = c * HC
            kd = pltpu.make_async_copy(
                k_ref.at[:, h0:h0 + HC, :, :], kq.at[c], dma_sems.at[2 * c])
            vd = pltpu.make_async_copy(
                v_ref.at[:, h0:h0 + HC, :, :], vq.at[c],
                dma_sems.at[2 * c + 1])
            kd.start()
            vd.start()
            dmas.append((kd, vd))

        z_off = my_z * P_loc
        btv = bt_ref[:, :]
        lensv = lens_ref[:, :]
        p_iota = lax.broadcasted_iota(jnp.int32, (B, NB, P_loc), 2) + z_off
        j_iota = lax.broadcasted_iota(jnp.int32, (B, NB, P_loc), 1)
        hit = (btv[:, :, None] == p_iota) & (j_iota < lensv[:, :, None])
        w_page = jnp.sum(hit.astype(jnp.float32), axis=1)
        lnw = jnp.where(w_page > 0, jnp.log(w_page), jnp.float32(-1e30))

        qT = jnp.transpose(q_ref[:, 0, :, :], (1, 0, 2)) * scale

        rdmas = []
        locals_ = []
        for c in range(N_CHUNK):
            h0 = c * HC
            kd, vd = dmas[c]
            kd.wait()
            vd.wait()
            qb = jnp.broadcast_to(
                qT[h0:h0 + HC][None], (BS, HC, B, D)).reshape(BS * HC, B, D)
            k_c = kq[c].reshape(BS * HC, D, P_loc)
            v_c = vq[c].reshape(BS * HC, D, P_loc)
            s3 = lax.dot_general(
                qb, k_c, (((2,), (1,)), ((0,), (0,))),
                preferred_element_type=jnp.float32)
            s4 = s3.reshape(BS, HC, B, P_loc) + lnw[None, None, :, :]
            m_c = jnp.max(jnp.max(s4, axis=0), axis=2)
            p4 = jnp.exp(s4 - m_c[None, :, :, None])
            l_c = jnp.sum(jnp.sum(p4, axis=0), axis=2)
            o3 = lax.dot_general(
                p4.reshape(BS * HC, B, P_loc), v_c,
                (((2,), (2,)), ((0,), (0,))),
                preferred_element_type=jnp.float32)
            o_c = jnp.sum(o3.reshape(BS, HC, B, D), axis=0)

            o_send[h0:h0 + HC, :, 0:D] = o_c
            o_send[h0:h0 + HC, :, D:D + 1] = m_c[:, :, None]
            o_send[h0:h0 + HC, :, D + 1:D + 2] = l_c[:, :, None]
            locals_.append((m_c, l_c, o_c))

            if c == 0:
                pl.semaphore_wait(barrier_sem, 1)
            rdma = pltpu.make_async_remote_copy(
                src_ref=o_send.at[h0:h0 + HC],
                dst_ref=o_recv.at[h0:h0 + HC],
                send_sem=send_sems.at[c], recv_sem=recv_sems.at[c],
                device_id=partner, device_id_type=pl.DeviceIdType.MESH)
            rdma.start()
            rdmas.append(rdma)

        for c in range(N_CHUNK):
            h0 = c * HC
            rdmas[c].wait_recv()
            m_l, l_l, o_l = locals_[c]
            m_r = o_recv[h0:h0 + HC, :, D]
            l_r = o_recv[h0:h0 + HC, :, D + 1]
            m_n = jnp.maximum(m_l, m_r)
            a_l = jnp.exp(m_l - m_n)
            a_r = jnp.exp(m_r - m_n)
            l_tot = a_l * l_l + a_r * l_r
            comb = (a_l[:, :, None] * o_l
                    + a_r[:, :, None] * o_recv[h0:h0 + HC, :, 0:D]) \
                / l_tot[:, :, None]
            out_ref[:, 0, h0:h0 + HC, :] = jnp.transpose(comb, (1, 0, 2))

        for rdma in rdmas:
            rdma.wait_send()

    K2 = jnp.transpose(K, (1, 2, 3, 0))
    V2 = jnp.transpose(V, (1, 2, 3, 0))

    return pl.pallas_call(
        body,
        out_shape=jax.ShapeDtypeStruct((B, QL, H, D), jnp.float32),
        in_specs=[
            pl.BlockSpec(memory_space=pltpu.VMEM),
            pl.BlockSpec(memory_space=pl.ANY),
            pl.BlockSpec(memory_space=pl.ANY),
            pl.BlockSpec(memory_space=pltpu.VMEM),
            pl.BlockSpec(memory_space=pltpu.VMEM),
        ],
        out_specs=pl.BlockSpec(memory_space=pltpu.VMEM),
        scratch_shapes=[
            pltpu.VMEM((N_CHUNK, BS, H // N_CHUNK, D, P_loc), jnp.float32),
            pltpu.VMEM((N_CHUNK, BS, H // N_CHUNK, D, P_loc), jnp.float32),
            pltpu.VMEM((H, B, D + 2), jnp.float32),
            pltpu.VMEM((H, B, D + 2), jnp.float32),
            pltpu.SemaphoreType.DMA((2 * N_CHUNK,)),
            pltpu.SemaphoreType.DMA((N_CHUNK,)),
            pltpu.SemaphoreType.DMA((N_CHUNK,)),
        ],
        compiler_params=pltpu.CompilerParams(
            collective_id=0, vmem_limit_bytes=48 * 1024 * 1024),
    )(Q, K2, V2, bt, lens.reshape(B, 1))
